# baseline (speedup 1.0000x reference)
import jax
import jax.numpy as jnp
from jax.experimental import pallas as pl

_EPSILON = 0.05
_SRC_ALPHA = 2.0
_TGT_ALPHA = 2.0
_EVENT_ALPHA = 2.0
_ROW_BLOCK = 64

_KEY_BASE = -1107296257
_KEY_BITS = 28


def _float_key(f):
    b = jax.lax.bitcast_convert_type(f, jnp.int32)
    return b ^ (jnp.right_shift(b, 31) & jnp.int32(0x7FFFFFFF))


def _neg_count_lt(x, cand):
    return jnp.sum(jax.lax.shift_right_arithmetic(x - cand, 31), axis=-1,
                   keepdims=True)


def _topk_thresholds(key, k):
    rows, n = key.shape

    def step(i, t):
        bit = jax.lax.shift_left(
            jnp.int32(1), 2 * (jnp.int32(_KEY_BITS // 2 - 1) - i))
        c1 = t + bit
        c2 = c1 + bit
        c3 = c2 + bit
        cnt1 = _neg_count_lt(key, c1) + n
        cnt2 = _neg_count_lt(key, c2) + n
        cnt3 = _neg_count_lt(key, c3) + n
        t = jnp.where(cnt1 >= k, c1, t)
        t = jnp.where(cnt2 >= k, c2, t)
        return jnp.where(cnt3 >= k, c3, t)

    t0 = jnp.full((rows, 1), _KEY_BASE, jnp.int32)
    t = jax.lax.fori_loop(0, _KEY_BITS // 2, step, t0)

    n_gt = _neg_count_lt(key, t + 1) + n
    m = k - n_gt

    idx = jax.lax.broadcasted_iota(jnp.int32, key.shape, 1)
    z = jnp.where(key == t, idx, jnp.int32(2 * n))

    def jstep(i, j):
        bit = jax.lax.shift_left(jnp.int32(1), jnp.int32(13) - i)
        cand = j + bit
        cnt = -_neg_count_lt(z, cand + 1)
        return jnp.where(cnt <= m, cand, j)

    j0 = jnp.full((rows, 1), jnp.int32(-1))
    j = jax.lax.fori_loop(0, 14, jstep, j0)
    return t, z <= j


def _body(uw_ref, ue_ref, ks_ref, kt_ref, dr_ref, src_ref, tgt_ref,
          drout_ref):
    rb = ue_ref.shape[0]
    ue = ue_ref[...]
    f_src = jnp.log(uw_ref[0]) + jnp.log(ue) * (1.0 / _EVENT_ALPHA)
    f_tgt = jnp.log(uw_ref[1]) + jnp.log(1.0 - ue) * (1.0 / _EVENT_ALPHA)
    key = jnp.concatenate([_float_key(f_src), _float_key(f_tgt)], axis=0)
    k = jnp.concatenate([ks_ref[:, :1], kt_ref[:, :1]], axis=0)
    t, tie_mask = _topk_thresholds(key, k)
    mask = (key > t) | tie_mask
    src_ref[...] = mask[:rb]
    tgt_ref[...] = mask[rb:]
    drout_ref[...] = jnp.broadcast_to(dr_ref[:, :1], drout_ref.shape)


def kernel(U_w, U_event, U_rate):
    b, n = U_event.shape
    lin = jnp.linspace(_EPSILON, 1.0 - _EPSILON, b)
    u = (lin + U_rate) % 1.0
    r_src = jnp.exp(jnp.log(u) / _SRC_ALPHA)
    r_tgt = jnp.exp(jnp.log(1.0 - u) / _TGT_ALPHA)
    dr = jnp.exp(jnp.log(u) * (1.0 / _SRC_ALPHA - 1.0)) / _SRC_ALPHA
    k_src = (r_src * n).astype(jnp.int32)
    k_tgt = (r_tgt * n).astype(jnp.int32)

    ks = jnp.broadcast_to(k_src[:, None], (b, 128))
    kt = jnp.broadcast_to(k_tgt[:, None], (b, 128))
    drb = jnp.broadcast_to(dr[:, None], (b, 128))

    rb = _ROW_BLOCK
    grid = (b // rb,)
    src, tgt, dr_out = pl.pallas_call(
        _body,
        grid=grid,
        in_specs=[
            pl.BlockSpec((2, rb, n), lambda i: (0, i, 0)),
            pl.BlockSpec((rb, n), lambda i: (i, 0)),
            pl.BlockSpec((rb, 128), lambda i: (i, 0)),
            pl.BlockSpec((rb, 128), lambda i: (i, 0)),
            pl.BlockSpec((rb, 128), lambda i: (i, 0)),
        ],
        out_specs=[
            pl.BlockSpec((rb, n), lambda i: (i, 0)),
            pl.BlockSpec((rb, n), lambda i: (i, 0)),
            pl.BlockSpec((rb, n), lambda i: (i, 0)),
        ],
        out_shape=[
            jax.ShapeDtypeStruct((b, n), jnp.bool_),
            jax.ShapeDtypeStruct((b, n), jnp.bool_),
            jax.ShapeDtypeStruct((b, n), jnp.float32),
        ],
    )(U_w, U_event, ks, kt, drb)
    return (src, tgt, dr_out)


# speedup vs baseline: 1.1127x; 1.1127x over previous
import jax
import jax.numpy as jnp
from jax.experimental import pallas as pl

_EPSILON = 0.05
_SRC_ALPHA = 2.0
_TGT_ALPHA = 2.0
_EVENT_ALPHA = 2.0
_ROW_BLOCK = 64

_KEY_BASE = -1107296257
_KEY_BITS = 28


def _float_key(f):
    b = jax.lax.bitcast_convert_type(f, jnp.int32)
    return b ^ (jnp.right_shift(b, 31) & jnp.int32(0x7FFFFFFF))


def _neg_count_lt(x, cand):
    return jnp.sum(jax.lax.shift_right_arithmetic(x - cand, 31), axis=-1,
                   keepdims=True)


def _topk_thresholds(key, k):
    rows, n = key.shape

    def step(i, t):
        bit = jax.lax.shift_left(jnp.int32(1), jnp.int32(_KEY_BITS - 1) - i)
        cand = t + bit
        cnt = _neg_count_lt(key, cand) + n
        return jnp.where(cnt >= k, cand, t)

    t0 = jnp.full((rows, 1), _KEY_BASE, jnp.int32)
    t = jax.lax.fori_loop(0, _KEY_BITS, step, t0)

    n_gt = _neg_count_lt(key, t + 1) + n
    m = k - n_gt

    idx = jax.lax.broadcasted_iota(jnp.int32, key.shape, 1)
    z = jnp.where(key == t, idx, jnp.int32(2 * n))

    def jstep(i, j):
        bit = jax.lax.shift_left(jnp.int32(1), jnp.int32(13) - i)
        cand = j + bit
        cnt = -_neg_count_lt(z, cand + 1)
        return jnp.where(cnt <= m, cand, j)

    j0 = jnp.full((rows, 1), jnp.int32(-1))
    j = jax.lax.fori_loop(0, 14, jstep, j0)
    return t, z <= j


def _body(uw_ref, ue_ref, ks_ref, kt_ref, dr_ref, src_ref, tgt_ref,
          drout_ref):
    rb = ue_ref.shape[0]
    ue = ue_ref[...]
    f_src = jnp.log(uw_ref[0]) + jnp.log(ue) * (1.0 / _EVENT_ALPHA)
    f_tgt = jnp.log(uw_ref[1]) + jnp.log(1.0 - ue) * (1.0 / _EVENT_ALPHA)
    key = jnp.concatenate([_float_key(f_src), _float_key(f_tgt)], axis=0)
    k = jnp.concatenate([ks_ref[:, :1], kt_ref[:, :1]], axis=0)
    t, tie_mask = _topk_thresholds(key, k)
    mask = (key > t) | tie_mask
    src_ref[...] = mask[:rb]
    tgt_ref[...] = mask[rb:]
    drout_ref[...] = jnp.broadcast_to(dr_ref[:, :1], drout_ref.shape)


def kernel(U_w, U_event, U_rate):
    b, n = U_event.shape
    lin = jnp.linspace(_EPSILON, 1.0 - _EPSILON, b)
    u = (lin + U_rate) % 1.0
    r_src = jnp.exp(jnp.log(u) / _SRC_ALPHA)
    r_tgt = jnp.exp(jnp.log(1.0 - u) / _TGT_ALPHA)
    dr = jnp.exp(jnp.log(u) * (1.0 / _SRC_ALPHA - 1.0)) / _SRC_ALPHA
    k_src = (r_src * n).astype(jnp.int32)
    k_tgt = (r_tgt * n).astype(jnp.int32)

    ks = jnp.broadcast_to(k_src[:, None], (b, 128))
    kt = jnp.broadcast_to(k_tgt[:, None], (b, 128))
    drb = jnp.broadcast_to(dr[:, None], (b, 128))

    rb = _ROW_BLOCK
    grid = (b // rb,)
    src, tgt, dr_out = pl.pallas_call(
        _body,
        grid=grid,
        in_specs=[
            pl.BlockSpec((2, rb, n), lambda i: (0, i, 0)),
            pl.BlockSpec((rb, n), lambda i: (i, 0)),
            pl.BlockSpec((rb, 128), lambda i: (i, 0)),
            pl.BlockSpec((rb, 128), lambda i: (i, 0)),
            pl.BlockSpec((rb, 128), lambda i: (i, 0)),
        ],
        out_specs=[
            pl.BlockSpec((rb, n), lambda i: (i, 0)),
            pl.BlockSpec((rb, n), lambda i: (i, 0)),
            pl.BlockSpec((rb, n), lambda i: (i, 0)),
        ],
        out_shape=[
            jax.ShapeDtypeStruct((b, n), jnp.bool_),
            jax.ShapeDtypeStruct((b, n), jnp.bool_),
            jax.ShapeDtypeStruct((b, n), jnp.float32),
        ],
    )(U_w, U_event, ks, kt, drb)
    return (src, tgt, dr_out)


# speedup vs baseline: 1.1421x; 1.0264x over previous
import jax
import jax.numpy as jnp
from jax.experimental import pallas as pl

_EPSILON = 0.05
_SRC_ALPHA = 2.0
_TGT_ALPHA = 2.0
_EVENT_ALPHA = 2.0
_ROW_BLOCK = 64

_KEY_BASE = -1107296257
_KEY_BITS = 28


def _float_key(f):
    b = jax.lax.bitcast_convert_type(f, jnp.int32)
    return b ^ (jnp.right_shift(b, 31) & jnp.int32(0x7FFFFFFF))


def _neg_count_lt(x, cand):
    return jnp.sum(jax.lax.shift_right_arithmetic(x - cand, 31), axis=-1,
                   keepdims=True)


def _topk_thresholds(key, k):
    rows, n = key.shape

    def step(i, t):
        bit = jax.lax.shift_left(jnp.int32(1), jnp.int32(_KEY_BITS - 1) - i)
        cand = t + bit
        cnt = _neg_count_lt(key, cand) + n
        return jnp.where(cnt >= k, cand, t)

    t0 = jnp.full((rows, 1), _KEY_BASE, jnp.int32)
    t = jax.lax.fori_loop(0, _KEY_BITS, step, t0, unroll=4)

    n_gt = _neg_count_lt(key, t + 1) + n
    m = k - n_gt

    idx = jax.lax.broadcasted_iota(jnp.int32, key.shape, 1)
    z = jnp.where(key == t, idx, jnp.int32(2 * n))

    def jstep(i, j):
        bit = jax.lax.shift_left(jnp.int32(1), jnp.int32(13) - i)
        cand = j + bit
        cnt = -_neg_count_lt(z, cand + 1)
        return jnp.where(cnt <= m, cand, j)

    j0 = jnp.full((rows, 1), jnp.int32(-1))
    j = jax.lax.fori_loop(0, 14, jstep, j0)
    return t, z <= j


def _body(uw_ref, ue_ref, ks_ref, kt_ref, dr_ref, src_ref, tgt_ref,
          drout_ref):
    rb = ue_ref.shape[0]
    ue = ue_ref[...]
    f_src = jnp.log(uw_ref[0]) + jnp.log(ue) * (1.0 / _EVENT_ALPHA)
    f_tgt = jnp.log(uw_ref[1]) + jnp.log(1.0 - ue) * (1.0 / _EVENT_ALPHA)
    key = jnp.concatenate([_float_key(f_src), _float_key(f_tgt)], axis=0)
    k = jnp.concatenate([ks_ref[:, :1], kt_ref[:, :1]], axis=0)
    t, tie_mask = _topk_thresholds(key, k)
    mask = (key > t) | tie_mask
    src_ref[...] = mask[:rb]
    tgt_ref[...] = mask[rb:]
    drout_ref[...] = jnp.broadcast_to(dr_ref[:, :1], drout_ref.shape)


def kernel(U_w, U_event, U_rate):
    b, n = U_event.shape
    lin = jnp.linspace(_EPSILON, 1.0 - _EPSILON, b)
    u = (lin + U_rate) % 1.0
    r_src = jnp.exp(jnp.log(u) / _SRC_ALPHA)
    r_tgt = jnp.exp(jnp.log(1.0 - u) / _TGT_ALPHA)
    dr = jnp.exp(jnp.log(u) * (1.0 / _SRC_ALPHA - 1.0)) / _SRC_ALPHA
    k_src = (r_src * n).astype(jnp.int32)
    k_tgt = (r_tgt * n).astype(jnp.int32)

    ks = jnp.broadcast_to(k_src[:, None], (b, 128))
    kt = jnp.broadcast_to(k_tgt[:, None], (b, 128))
    drb = jnp.broadcast_to(dr[:, None], (b, 128))

    rb = _ROW_BLOCK
    grid = (b // rb,)
    src, tgt, dr_out = pl.pallas_call(
        _body,
        grid=grid,
        in_specs=[
            pl.BlockSpec((2, rb, n), lambda i: (0, i, 0)),
            pl.BlockSpec((rb, n), lambda i: (i, 0)),
            pl.BlockSpec((rb, 128), lambda i: (i, 0)),
            pl.BlockSpec((rb, 128), lambda i: (i, 0)),
            pl.BlockSpec((rb, 128), lambda i: (i, 0)),
        ],
        out_specs=[
            pl.BlockSpec((rb, n), lambda i: (i, 0)),
            pl.BlockSpec((rb, n), lambda i: (i, 0)),
            pl.BlockSpec((rb, n), lambda i: (i, 0)),
        ],
        out_shape=[
            jax.ShapeDtypeStruct((b, n), jnp.bool_),
            jax.ShapeDtypeStruct((b, n), jnp.bool_),
            jax.ShapeDtypeStruct((b, n), jnp.float32),
        ],
    )(U_w, U_event, ks, kt, drb)
    return (src, tgt, dr_out)
